# baseline (speedup 1.0000x reference)
import functools
import math

import numpy as np
import jax
import jax.numpy as jnp
from jax import lax
from jax.experimental import pallas as pl
from jax.experimental.pallas import tpu as pltpu
from jax.experimental.pallas import tpu_sc as plsc

_N = 32768
_C = 1024
_K = 32
_RADIUS = 0.1
_NF = 10
_LANES = 128
_ROWS = _N // _LANES

_RB = 2048
_NB = _N // _RB


def _ball_thresh_sq():
    r = np.float32(_RADIUS)
    t = np.float32(r * r)
    inf32 = np.float32(np.inf)
    z32 = np.float32(0.0)
    while np.sqrt(np.nextafter(t, inf32)) <= r:
        t = np.nextafter(t, inf32)
    while np.sqrt(t) > r:
        t = np.nextafter(t, z32)
    return float(t)


_T2 = _ball_thresh_sq()


def _fps_body(px_ref, py_ref, samp_ref):
    px = px_ref[...]
    py = py_ref[...]
    lin = (lax.broadcasted_iota(jnp.int32, px.shape, 0) * _LANES
           + lax.broadcasted_iota(jnp.int32, px.shape, 1))
    samp_ref[0] = 0
    cx0 = jnp.sum(jnp.where(lin == 0, px, 0.0))
    cy0 = jnp.sum(jnp.where(lin == 0, py, 0.0))
    dists0 = jnp.full(px.shape, jnp.inf, dtype=jnp.float32)

    def body(i, carry):
        d2s, cx, cy = carry
        dx = px - cx
        dy = py - cy
        nd2 = dx * dx + dy * dy
        d2s = jnp.minimum(d2s, nd2)
        m2 = jnp.max(d2s, axis=(0, 1), keepdims=True)
        s = jnp.sqrt(m2)
        sp = lax.bitcast_convert_type(
            lax.bitcast_convert_type(s, jnp.int32) - 1, jnp.float32)
        z0b = lax.bitcast_convert_type(sp * sp, jnp.int32)
        offs = lax.broadcasted_iota(jnp.int32, (16, 1), 0) - 8
        zc = lax.bitcast_convert_type(z0b + offs, jnp.float32)
        ok = jnp.sqrt(zc) <= sp
        z = jnp.max(jnp.where(ok, zc, -1.0), axis=0, keepdims=True)
        z = jnp.where(m2 > 0.0, z, -1.0)
        mask = d2s > z
        nxt = jnp.min(jnp.where(mask, lin, jnp.int32(2**30)))
        cnt = jnp.sum(mask.astype(jnp.int32))
        sx = jnp.sum(jnp.where(mask, px, 0.0))
        sy = jnp.sum(jnp.where(mask, py, 0.0))
        samp_ref[i] = nxt

        def _tie(_):
            sel = lin == nxt
            return (jnp.sum(jnp.where(sel, px, 0.0)),
                    jnp.sum(jnp.where(sel, py, 0.0)))

        ncx, ncy = lax.cond(cnt == 1, lambda _: (sx, sy), _tie, 0)
        return (d2s, ncx, ncy)

    lax.fori_loop(1, _C, body, (dists0, cx0, cy0))


def _fps(px, py):
    return pl.pallas_call(
        _fps_body,
        out_shape=jax.ShapeDtypeStruct((_C,), jnp.int32),
        in_specs=[pl.BlockSpec(memory_space=pltpu.VMEM),
                  pl.BlockSpec(memory_space=pltpu.VMEM)],
        out_specs=pl.BlockSpec(memory_space=pltpu.SMEM),
    )(px, py)


_NWORK = 32
_CPW = _C // _NWORK
_CHUNKS = _N // 16


def _ballq_sc_body(px_hbm, py_hbm, samp_hbm, relx_hbm, rely_hbm, cxo_hbm,
                   cyo_hbm, pxv, pyv, sampv, cxv, cyv, cbuf, rxv, ryv):
    wid = lax.axis_index("s") * 2 + lax.axis_index("c")
    base = wid * _CPW
    pltpu.sync_copy(px_hbm, pxv)
    pltpu.sync_copy(py_hbm, pyv)
    pltpu.sync_copy(samp_hbm.at[pl.ds(base, _CPW)], sampv)
    lane = lax.iota(jnp.int32, 16)

    for j in range(_CPW // 16):
        sidx = sampv[pl.ds(j * 16, 16)]
        cxv[pl.ds(j * 16, 16)] = plsc.load_gather(pxv, [sidx])
        cyv[pl.ds(j * 16, 16)] = plsc.load_gather(pyv, [sidx])
    pltpu.sync_copy(cxv, cxo_hbm.at[pl.ds(base, _CPW)])
    pltpu.sync_copy(cyv, cyo_hbm.at[pl.ds(base, _CPW)])

    thr = jnp.float32(_T2)

    def per_centroid(c, _):
        cvec = jnp.full((16,), c, dtype=jnp.int32)
        cxs = plsc.load_gather(cxv, [cvec])
        cys = plsc.load_gather(cyv, [cvec])

        def cond(st):
            chunk, wptr = st
            return (wptr < _K) & (chunk < _CHUNKS)

        def step(st):
            chunk, wptr = st
            xx = pxv[pl.ds(chunk * 16, 16)]
            yy = pyv[pl.ds(chunk * 16, 16)]
            dx = xx - cxs
            dy = yy - cys
            d2 = dx * dx + dy * dy
            msk = d2 <= thr
            plsc.store_compressed(cbuf.at[pl.ds(wptr, 16)],
                                  chunk * 16 + lane, mask=msk)
            cnt = jnp.sum(msk.astype(jnp.int32))
            return (chunk + jnp.int32(1), wptr + cnt)

        _, wptr = lax.while_loop(cond, step, (jnp.int32(0), jnp.int32(0)))

        zvec = jnp.full((16,), 0, dtype=jnp.int32)
        first = plsc.load_gather(cbuf, [zvec])
        for j in range(_K // 16):
            kio = j * 16 + lane
            vals = cbuf[pl.ds(j * 16, 16)]
            vals = jnp.where(kio < wptr, vals, first)
            gx = plsc.load_gather(pxv, [vals])
            gy = plsc.load_gather(pyv, [vals])
            rxv[pl.ds(c * _K + j * 16, 16)] = gx - cxs
            ryv[pl.ds(c * _K + j * 16, 16)] = gy - cys
        return 0

    lax.fori_loop(0, _CPW, per_centroid, 0)
    pltpu.sync_copy(rxv, relx_hbm.at[pl.ds(base * _K, _CPW * _K)])
    pltpu.sync_copy(ryv, rely_hbm.at[pl.ds(base * _K, _CPW * _K)])


@functools.cache
def _ballq_sc():
    return pl.kernel(
        _ballq_sc_body,
        out_type=[
            jax.ShapeDtypeStruct((_N,), jnp.float32),
            jax.ShapeDtypeStruct((_N,), jnp.float32),
            jax.ShapeDtypeStruct((_C,), jnp.float32),
            jax.ShapeDtypeStruct((_C,), jnp.float32),
        ],
        mesh=plsc.VectorSubcoreMesh(core_axis_name="c", subcore_axis_name="s"),
        compiler_params=pltpu.CompilerParams(needs_layout_passes=False),
        scratch_types=[
            pltpu.VMEM((_N,), jnp.float32),
            pltpu.VMEM((_N,), jnp.float32),
            pltpu.VMEM((_CPW,), jnp.int32),
            pltpu.VMEM((_CPW,), jnp.float32),
            pltpu.VMEM((_CPW,), jnp.float32),
            pltpu.VMEM((48,), jnp.int32),
            pltpu.VMEM((_CPW * _K,), jnp.float32),
            pltpu.VMEM((_CPW * _K,), jnp.float32),
        ],
    )


def _mlp_body(rx_ref, ry_ref, w1_ref, b1_ref, g1_ref, be1_ref,
              w2_ref, b2_ref, g2_ref, be2_ref, w3_ref, b3_ref,
              out_ref, h1_ref, h2_ref):
    rows = lax.broadcasted_iota(jnp.int32, (4 * _NF, 1), 0)
    pow2 = lax.shift_left(jnp.int32(1), rows // 4).astype(jnp.float32)
    fmul = pow2 * jnp.float32(math.pi)
    ymask = ((rows // 2) % 2) == 1
    cmask = (rows % 2) == 1
    w1t = w1_ref[...]
    b1 = b1_ref[...]

    def seg_max(x):
        for s in (16, 8, 4, 2, 1):
            x = jnp.maximum(x, pltpu.roll(x, _RB - s, 1))
        return x

    def enc_block(b, acc):
        s1, ss1 = acc
        rx = rx_ref[:, pl.ds(b * _RB, _RB)]
        ry = ry_ref[:, pl.ds(b * _RB, _RB)]
        rel = jnp.where(ymask, ry, rx)
        ang = rel * fmul
        enc = jnp.where(cmask, jnp.cos(ang), jnp.sin(ang))
        h1p = jnp.dot(w1t, enc, preferred_element_type=jnp.float32) + b1
        h1_ref[:, pl.ds(b * _RB, _RB)] = h1p
        return (s1 + jnp.sum(h1p, axis=1, keepdims=True),
                ss1 + jnp.sum(h1p * h1p, axis=1, keepdims=True))

    z64 = jnp.zeros((64, 1), jnp.float32)
    s1, ss1 = lax.fori_loop(0, _NB, enc_block, (z64, z64))
    mu1 = s1 / _N
    var1 = ss1 / _N - mu1 * mu1
    den1 = jnp.sqrt(var1 + 1e-5)
    g1 = g1_ref[...]
    be1 = be1_ref[...]
    w2t = w2_ref[...]
    b2 = b2_ref[...]

    def l2_block(b, acc):
        s2, ss2 = acc
        h1p = h1_ref[:, pl.ds(b * _RB, _RB)]
        h1 = jax.nn.relu((h1p - mu1) / den1 * g1 + be1)
        h2p = jnp.dot(w2t, h1, preferred_element_type=jnp.float32) + b2
        h2_ref[:, pl.ds(b * _RB, _RB)] = h2p
        return (s2 + jnp.sum(h2p, axis=1, keepdims=True),
                ss2 + jnp.sum(h2p * h2p, axis=1, keepdims=True))

    z128 = jnp.zeros((128, 1), jnp.float32)
    s2, ss2 = lax.fori_loop(0, _NB, l2_block, (z128, z128))
    mu2 = s2 / _N
    var2 = ss2 / _N - mu2 * mu2
    den2 = jnp.sqrt(var2 + 1e-5)
    g2 = g2_ref[...]
    be2 = be2_ref[...]
    w3t = w3_ref[...]
    b3 = b3_ref[...]
    gsel = (lax.broadcasted_iota(jnp.int32, (_RB, _RB // _K), 0)
            == _K * lax.broadcasted_iota(jnp.int32, (_RB, _RB // _K), 1)
            ).astype(jnp.float32)

    def l3_block(b, _):
        h2p = h2_ref[:, pl.ds(b * _RB, _RB)]
        h2 = jax.nn.relu((h2p - mu2) / den2 * g2 + be2)
        h3 = jnp.dot(w3t, h2, preferred_element_type=jnp.float32) + b3
        gmax = seg_max(h3)
        out_ref[b, :, :] = jnp.dot(
            gmax, gsel, preferred_element_type=jnp.float32)
        return 0

    lax.fori_loop(0, _NB, l3_block, 0)


def _mlp(rx, ry, W1, b1, g1, be1, W2, b2, g2, be2, W3, b3):
    vmem = pl.BlockSpec(memory_space=pltpu.VMEM)
    return pl.pallas_call(
        _mlp_body,
        out_shape=jax.ShapeDtypeStruct((_NB, 16, _RB // _K), jnp.float32),
        in_specs=[vmem] * 12,
        out_specs=vmem,
        scratch_shapes=[
            pltpu.VMEM((64, _N), jnp.float32),
            pltpu.VMEM((128, _N), jnp.float32),
        ],
    )(rx, ry, W1.T, b1.reshape(64, 1), g1.reshape(64, 1), be1.reshape(64, 1),
      W2.T, b2.reshape(128, 1), g2.reshape(128, 1), be2.reshape(128, 1),
      W3.T, b3.reshape(16, 1))


def kernel(points, W1, b1, g1, be1, W2, b2, g2, be2, W3, b3):
    pxf = points[:, 0]
    pyf = points[:, 1]
    px = pxf.reshape(_ROWS, _LANES)
    py = pyf.reshape(_ROWS, _LANES)
    samp = _fps(px, py)
    relx, rely, cx, cy = _ballq_sc()(pxf, pyf, samp)
    feats3 = _mlp(relx.reshape(1, _N), rely.reshape(1, _N),
                  W1, b1, g1, be1, W2, b2, g2, be2, W3, b3)
    feats = feats3.transpose(0, 2, 1).reshape(_C, 16)
    centroids = jnp.stack([cx, cy], axis=-1)
    return (feats, centroids)


# speedup vs baseline: 1.4671x; 1.4671x over previous
import functools
import math

import numpy as np
import jax
import jax.numpy as jnp
from jax import lax
from jax.experimental import pallas as pl
from jax.experimental.pallas import tpu as pltpu
from jax.experimental.pallas import tpu_sc as plsc

_N = 32768
_C = 1024
_K = 32
_RADIUS = 0.1
_NF = 10
_LANES = 128
_ROWS = _N // _LANES

_RB = 2048
_NB = _N // _RB


def _ball_thresh_sq():
    r = np.float32(_RADIUS)
    t = np.float32(r * r)
    inf32 = np.float32(np.inf)
    z32 = np.float32(0.0)
    while np.sqrt(np.nextafter(t, inf32)) <= r:
        t = np.nextafter(t, inf32)
    while np.sqrt(t) > r:
        t = np.nextafter(t, z32)
    return float(t)


_T2 = _ball_thresh_sq()


def _fps_body(px_ref, py_ref, cx_ref, cy_ref):
    px = px_ref[...]
    py = py_ref[...]
    cx0 = px_ref[0, 0]
    cy0 = py_ref[0, 0]
    cx_ref[0] = cx0
    cy_ref[0] = cy0
    dists0 = jnp.full(px.shape, jnp.inf, dtype=jnp.float32)

    def body(i, carry):
        dists, cx, cy = carry
        dx = px - cx
        dy = py - cy
        nd = jnp.sqrt(dx * dx + dy * dy)
        dists = jnp.minimum(dists, nd)
        m = jnp.max(dists)
        mask = dists == m
        cnt = jnp.sum(mask.astype(jnp.int32))
        sx = jnp.sum(jnp.where(mask, px, 0.0))
        sy = jnp.sum(jnp.where(mask, py, 0.0))

        def _tie(_):
            lin = (lax.broadcasted_iota(jnp.int32, px.shape, 0) * _LANES
                   + lax.broadcasted_iota(jnp.int32, px.shape, 1))
            nxt = jnp.min(jnp.where(mask, lin, jnp.int32(2**30)))
            sel = lin == nxt
            return (jnp.sum(jnp.where(sel, px, 0.0)),
                    jnp.sum(jnp.where(sel, py, 0.0)))

        ncx, ncy = lax.cond(cnt == 1, lambda _: (sx, sy), _tie, 0)
        cx_ref[i] = ncx
        cy_ref[i] = ncy
        return (dists, ncx, ncy)

    lax.fori_loop(1, _C, body, (dists0, cx0, cy0))


def _fps(px, py):
    return pl.pallas_call(
        _fps_body,
        out_shape=[jax.ShapeDtypeStruct((_C,), jnp.float32),
                   jax.ShapeDtypeStruct((_C,), jnp.float32)],
        in_specs=[pl.BlockSpec(memory_space=pltpu.VMEM),
                  pl.BlockSpec(memory_space=pltpu.VMEM)],
        out_specs=[pl.BlockSpec(memory_space=pltpu.SMEM),
                   pl.BlockSpec(memory_space=pltpu.SMEM)],
    )(px, py)


_NWORK = 32
_CPW = _C // _NWORK
_CHUNKS = _N // 16


def _ballq_sc_body(px_hbm, py_hbm, cx_hbm, cy_hbm, relx_hbm, rely_hbm,
                   pxv, pyv, cxv, cyv, cbuf, rxv, ryv):
    wid = lax.axis_index("s") * 2 + lax.axis_index("c")
    base = wid * _CPW
    pltpu.sync_copy(px_hbm, pxv)
    pltpu.sync_copy(py_hbm, pyv)
    pltpu.sync_copy(cx_hbm.at[pl.ds(base, _CPW)], cxv)
    pltpu.sync_copy(cy_hbm.at[pl.ds(base, _CPW)], cyv)
    lane = lax.iota(jnp.int32, 16)

    thr = jnp.float32(_T2)

    def per_centroid(c, _):
        cvec = jnp.full((16,), c, dtype=jnp.int32)
        cxs = plsc.load_gather(cxv, [cvec])
        cys = plsc.load_gather(cyv, [cvec])

        def cond(st):
            pos, wptr = st
            return (wptr < _K) & (pos < _N)

        def step(st):
            pos, wptr = st
            for u in range(2):
                xx = pxv[pl.ds(pos + u * 16, 16)]
                yy = pyv[pl.ds(pos + u * 16, 16)]
                dx = xx - cxs
                dy = yy - cys
                d2 = dx * dx + dy * dy
                msk = d2 <= thr
                plsc.store_compressed(cbuf.at[pl.ds(wptr, 16)],
                                      pos + u * 16 + lane, mask=msk)
                wptr = wptr + jnp.sum(msk.astype(jnp.int32))
            return (pos + jnp.int32(32), wptr)

        _, wptr = lax.while_loop(cond, step, (jnp.int32(0), jnp.int32(0)))

        zvec = jnp.full((16,), 0, dtype=jnp.int32)
        first = plsc.load_gather(cbuf, [zvec])
        for j in range(_K // 16):
            kio = j * 16 + lane
            vals = cbuf[pl.ds(j * 16, 16)]
            vals = jnp.where(kio < wptr, vals, first)
            gx = plsc.load_gather(pxv, [vals])
            gy = plsc.load_gather(pyv, [vals])
            rxv[pl.ds(c * _K + j * 16, 16)] = gx - cxs
            ryv[pl.ds(c * _K + j * 16, 16)] = gy - cys
        return 0

    lax.fori_loop(0, _CPW, per_centroid, 0)
    pltpu.sync_copy(rxv, relx_hbm.at[pl.ds(base * _K, _CPW * _K)])
    pltpu.sync_copy(ryv, rely_hbm.at[pl.ds(base * _K, _CPW * _K)])


@functools.cache
def _ballq_sc():
    return pl.kernel(
        _ballq_sc_body,
        out_type=[
            jax.ShapeDtypeStruct((_N,), jnp.float32),
            jax.ShapeDtypeStruct((_N,), jnp.float32),
        ],
        mesh=plsc.VectorSubcoreMesh(core_axis_name="c", subcore_axis_name="s"),
        compiler_params=pltpu.CompilerParams(needs_layout_passes=False),
        scratch_types=[
            pltpu.VMEM((_N,), jnp.float32),
            pltpu.VMEM((_N,), jnp.float32),
            pltpu.VMEM((_CPW,), jnp.float32),
            pltpu.VMEM((_CPW,), jnp.float32),
            pltpu.VMEM((64,), jnp.int32),
            pltpu.VMEM((_CPW * _K,), jnp.float32),
            pltpu.VMEM((_CPW * _K,), jnp.float32),
        ],
    )


def _mlp_body(rx_ref, ry_ref, w1_ref, b1_ref, g1_ref, be1_ref,
              w2_ref, b2_ref, g2_ref, be2_ref, w3_ref, b3_ref,
              out_ref, h1_ref, h2_ref):
    rows = lax.broadcasted_iota(jnp.int32, (4 * _NF, 1), 0)
    pow2 = lax.shift_left(jnp.int32(1), rows // 4).astype(jnp.float32)
    fmul = pow2 * jnp.float32(math.pi)
    ymask = ((rows // 2) % 2) == 1
    cmask = (rows % 2) == 1
    w1t = w1_ref[...]
    b1 = b1_ref[...]

    def seg_max(x):
        for s in (16, 8, 4, 2, 1):
            x = jnp.maximum(x, pltpu.roll(x, _RB - s, 1))
        return x

    def enc_block(b, acc):
        s1, ss1 = acc
        rx = rx_ref[:, pl.ds(b * _RB, _RB)]
        ry = ry_ref[:, pl.ds(b * _RB, _RB)]
        rel = jnp.where(ymask, ry, rx)
        ang = rel * fmul
        enc = jnp.where(cmask, jnp.cos(ang), jnp.sin(ang))
        h1p = jnp.dot(w1t, enc, preferred_element_type=jnp.float32) + b1
        h1_ref[:, pl.ds(b * _RB, _RB)] = h1p
        return (s1 + jnp.sum(h1p, axis=1, keepdims=True),
                ss1 + jnp.sum(h1p * h1p, axis=1, keepdims=True))

    z64 = jnp.zeros((64, 1), jnp.float32)
    s1, ss1 = lax.fori_loop(0, _NB, enc_block, (z64, z64))
    mu1 = s1 / _N
    var1 = ss1 / _N - mu1 * mu1
    den1 = jnp.sqrt(var1 + 1e-5)
    g1 = g1_ref[...]
    be1 = be1_ref[...]
    w2t = w2_ref[...]
    b2 = b2_ref[...]

    def l2_block(b, acc):
        s2, ss2 = acc
        h1p = h1_ref[:, pl.ds(b * _RB, _RB)]
        h1 = jax.nn.relu((h1p - mu1) / den1 * g1 + be1)
        h2p = jnp.dot(w2t, h1, preferred_element_type=jnp.float32) + b2
        h2_ref[:, pl.ds(b * _RB, _RB)] = h2p
        return (s2 + jnp.sum(h2p, axis=1, keepdims=True),
                ss2 + jnp.sum(h2p * h2p, axis=1, keepdims=True))

    z128 = jnp.zeros((128, 1), jnp.float32)
    s2, ss2 = lax.fori_loop(0, _NB, l2_block, (z128, z128))
    mu2 = s2 / _N
    var2 = ss2 / _N - mu2 * mu2
    den2 = jnp.sqrt(var2 + 1e-5)
    g2 = g2_ref[...]
    be2 = be2_ref[...]
    w3t = w3_ref[...]
    b3 = b3_ref[...]
    gsel = (lax.broadcasted_iota(jnp.int32, (_RB, _RB // _K), 0)
            == _K * lax.broadcasted_iota(jnp.int32, (_RB, _RB // _K), 1)
            ).astype(jnp.float32)

    def l3_block(b, _):
        h2p = h2_ref[:, pl.ds(b * _RB, _RB)]
        h2 = jax.nn.relu((h2p - mu2) / den2 * g2 + be2)
        h3 = jnp.dot(w3t, h2, preferred_element_type=jnp.float32) + b3
        gmax = seg_max(h3)
        out_ref[b, :, :] = jnp.dot(
            gmax, gsel, preferred_element_type=jnp.float32)
        return 0

    lax.fori_loop(0, _NB, l3_block, 0)


def _mlp(rx, ry, W1, b1, g1, be1, W2, b2, g2, be2, W3, b3):
    vmem = pl.BlockSpec(memory_space=pltpu.VMEM)
    return pl.pallas_call(
        _mlp_body,
        out_shape=jax.ShapeDtypeStruct((_NB, 16, _RB // _K), jnp.float32),
        in_specs=[vmem] * 12,
        out_specs=vmem,
        scratch_shapes=[
            pltpu.VMEM((64, _N), jnp.float32),
            pltpu.VMEM((128, _N), jnp.float32),
        ],
    )(rx, ry, W1.T, b1.reshape(64, 1), g1.reshape(64, 1), be1.reshape(64, 1),
      W2.T, b2.reshape(128, 1), g2.reshape(128, 1), be2.reshape(128, 1),
      W3.T, b3.reshape(16, 1))


def kernel(points, W1, b1, g1, be1, W2, b2, g2, be2, W3, b3):
    pxf = points[:, 0]
    pyf = points[:, 1]
    px = pxf.reshape(_ROWS, _LANES)
    py = pyf.reshape(_ROWS, _LANES)
    cx, cy = _fps(px, py)
    relx, rely = _ballq_sc()(pxf, pyf, cx, cy)
    feats3 = _mlp(relx.reshape(1, _N), rely.reshape(1, _N),
                  W1, b1, g1, be1, W2, b2, g2, be2, W3, b3)
    feats = feats3.transpose(0, 2, 1).reshape(_C, 16)
    centroids = jnp.stack([cx, cy], axis=-1)
    return (feats, centroids)


# speedup vs baseline: 1.5146x; 1.0324x over previous
import functools
import math

import numpy as np
import jax
import jax.numpy as jnp
from jax import lax
from jax.experimental import pallas as pl
from jax.experimental.pallas import tpu as pltpu
from jax.experimental.pallas import tpu_sc as plsc

_N = 32768
_C = 1024
_K = 32
_RADIUS = 0.1
_NF = 10
_LANES = 128
_ROWS = _N // _LANES

_RB = 2048
_NB = _N // _RB


def _ball_thresh_sq():
    r = np.float32(_RADIUS)
    t = np.float32(r * r)
    inf32 = np.float32(np.inf)
    z32 = np.float32(0.0)
    while np.sqrt(np.nextafter(t, inf32)) <= r:
        t = np.nextafter(t, inf32)
    while np.sqrt(t) > r:
        t = np.nextafter(t, z32)
    return float(t)


_T2 = _ball_thresh_sq()


def _fps_body(px_ref, py_ref, cx_ref, cy_ref):
    px = px_ref[...]
    py = py_ref[...]
    cx0 = px_ref[0, 0]
    cy0 = py_ref[0, 0]
    cx_ref[0] = cx0
    cy_ref[0] = cy0
    dists0 = jnp.full(px.shape, jnp.inf, dtype=jnp.float32)

    def body(i, carry):
        dists, cx, cy = carry
        dx = px - cx
        dy = py - cy
        nd = jnp.sqrt(dx * dx + dy * dy)
        dists = jnp.minimum(dists, nd)
        m = jnp.max(dists)
        mask = dists == m
        cnt = jnp.sum(mask.astype(jnp.int32))
        sx = jnp.sum(jnp.where(mask, px, 0.0))
        sy = jnp.sum(jnp.where(mask, py, 0.0))

        def _tie(_):
            lin = (lax.broadcasted_iota(jnp.int32, px.shape, 0) * _LANES
                   + lax.broadcasted_iota(jnp.int32, px.shape, 1))
            nxt = jnp.min(jnp.where(mask, lin, jnp.int32(2**30)))
            sel = lin == nxt
            return (jnp.sum(jnp.where(sel, px, 0.0)),
                    jnp.sum(jnp.where(sel, py, 0.0)))

        ncx, ncy = lax.cond(cnt == 1, lambda _: (sx, sy), _tie, 0)
        cx_ref[i] = ncx
        cy_ref[i] = ncy
        return (dists, ncx, ncy)

    lax.fori_loop(1, _C, body, (dists0, cx0, cy0))


def _fps(px, py):
    return pl.pallas_call(
        _fps_body,
        out_shape=[jax.ShapeDtypeStruct((_C,), jnp.float32),
                   jax.ShapeDtypeStruct((_C,), jnp.float32)],
        in_specs=[pl.BlockSpec(memory_space=pltpu.VMEM),
                  pl.BlockSpec(memory_space=pltpu.VMEM)],
        out_specs=[pl.BlockSpec(memory_space=pltpu.SMEM),
                   pl.BlockSpec(memory_space=pltpu.SMEM)],
    )(px, py)


_NWORK = 32
_CPW = _C // _NWORK
_CHUNKS = _N // 16


def _ballq_sc_body(px_hbm, py_hbm, cx_hbm, cy_hbm, relx_hbm, rely_hbm,
                   pxv, pyv, cxv, cyv, cbuf, rxv, ryv):
    wid = lax.axis_index("s") * 2 + lax.axis_index("c")
    base = wid * _CPW
    pltpu.sync_copy(px_hbm, pxv)
    pltpu.sync_copy(py_hbm, pyv)
    pltpu.sync_copy(cx_hbm.at[pl.ds(base, _CPW)], cxv)
    pltpu.sync_copy(cy_hbm.at[pl.ds(base, _CPW)], cyv)
    lane = lax.iota(jnp.int32, 16)

    thr = jnp.float32(_T2)

    def per_centroid(c, _):
        cvec = jnp.full((16,), c, dtype=jnp.int32)
        cxs = plsc.load_gather(cxv, [cvec])
        cys = plsc.load_gather(cyv, [cvec])

        def cond(st):
            pos, wptr = st
            return (wptr < _K) & (pos < _N)

        def step(st):
            pos, wptr = st
            for u in range(4):
                xx = pxv[pl.ds(pos + u * 16, 16)]
                yy = pyv[pl.ds(pos + u * 16, 16)]
                dx = xx - cxs
                dy = yy - cys
                d2 = dx * dx + dy * dy
                msk = d2 <= thr
                plsc.store_compressed(cbuf.at[pl.ds(wptr, 16)],
                                      pos + u * 16 + lane, mask=msk)
                wptr = wptr + jnp.sum(msk.astype(jnp.int32))
            return (pos + jnp.int32(64), wptr)

        _, wptr = lax.while_loop(cond, step, (jnp.int32(0), jnp.int32(0)))

        zvec = jnp.full((16,), 0, dtype=jnp.int32)
        first = plsc.load_gather(cbuf, [zvec])
        for j in range(_K // 16):
            kio = j * 16 + lane
            vals = cbuf[pl.ds(j * 16, 16)]
            vals = jnp.where(kio < wptr, vals, first)
            gx = plsc.load_gather(pxv, [vals])
            gy = plsc.load_gather(pyv, [vals])
            rxv[pl.ds(c * _K + j * 16, 16)] = gx - cxs
            ryv[pl.ds(c * _K + j * 16, 16)] = gy - cys
        return 0

    lax.fori_loop(0, _CPW, per_centroid, 0)
    pltpu.sync_copy(rxv, relx_hbm.at[pl.ds(base * _K, _CPW * _K)])
    pltpu.sync_copy(ryv, rely_hbm.at[pl.ds(base * _K, _CPW * _K)])


@functools.cache
def _ballq_sc():
    return pl.kernel(
        _ballq_sc_body,
        out_type=[
            jax.ShapeDtypeStruct((_N,), jnp.float32),
            jax.ShapeDtypeStruct((_N,), jnp.float32),
        ],
        mesh=plsc.VectorSubcoreMesh(core_axis_name="c", subcore_axis_name="s"),
        compiler_params=pltpu.CompilerParams(needs_layout_passes=False),
        scratch_types=[
            pltpu.VMEM((_N,), jnp.float32),
            pltpu.VMEM((_N,), jnp.float32),
            pltpu.VMEM((_CPW,), jnp.float32),
            pltpu.VMEM((_CPW,), jnp.float32),
            pltpu.VMEM((96,), jnp.int32),
            pltpu.VMEM((_CPW * _K,), jnp.float32),
            pltpu.VMEM((_CPW * _K,), jnp.float32),
        ],
    )


def _mlp_body(rx_ref, ry_ref, w1_ref, b1_ref, g1_ref, be1_ref,
              w2_ref, b2_ref, g2_ref, be2_ref, w3_ref, b3_ref,
              out_ref, h1_ref, h2_ref):
    rows = lax.broadcasted_iota(jnp.int32, (4 * _NF, 1), 0)
    pow2 = lax.shift_left(jnp.int32(1), rows // 4).astype(jnp.float32)
    fmul = pow2 * jnp.float32(math.pi)
    ymask = ((rows // 2) % 2) == 1
    cmask = (rows % 2) == 1
    w1t = w1_ref[...]
    b1 = b1_ref[...]

    def seg_max(x):
        for s in (16, 8, 4, 2, 1):
            x = jnp.maximum(x, pltpu.roll(x, _RB - s, 1))
        return x

    def enc_block(b, acc):
        s1, ss1 = acc
        rx = rx_ref[:, pl.ds(b * _RB, _RB)]
        ry = ry_ref[:, pl.ds(b * _RB, _RB)]
        rel = jnp.where(ymask, ry, rx)
        ang = rel * fmul
        enc = jnp.where(cmask, jnp.cos(ang), jnp.sin(ang))
        h1p = jnp.dot(w1t, enc, preferred_element_type=jnp.float32) + b1
        h1_ref[:, pl.ds(b * _RB, _RB)] = h1p
        return (s1 + jnp.sum(h1p, axis=1, keepdims=True),
                ss1 + jnp.sum(h1p * h1p, axis=1, keepdims=True))

    z64 = jnp.zeros((64, 1), jnp.float32)
    s1, ss1 = lax.fori_loop(0, _NB, enc_block, (z64, z64))
    mu1 = s1 / _N
    var1 = ss1 / _N - mu1 * mu1
    den1 = jnp.sqrt(var1 + 1e-5)
    g1 = g1_ref[...]
    be1 = be1_ref[...]
    w2t = w2_ref[...]
    b2 = b2_ref[...]

    def l2_block(b, acc):
        s2, ss2 = acc
        h1p = h1_ref[:, pl.ds(b * _RB, _RB)]
        h1 = jax.nn.relu((h1p - mu1) / den1 * g1 + be1)
        h2p = jnp.dot(w2t, h1, preferred_element_type=jnp.float32) + b2
        h2_ref[:, pl.ds(b * _RB, _RB)] = h2p
        return (s2 + jnp.sum(h2p, axis=1, keepdims=True),
                ss2 + jnp.sum(h2p * h2p, axis=1, keepdims=True))

    z128 = jnp.zeros((128, 1), jnp.float32)
    s2, ss2 = lax.fori_loop(0, _NB, l2_block, (z128, z128))
    mu2 = s2 / _N
    var2 = ss2 / _N - mu2 * mu2
    den2 = jnp.sqrt(var2 + 1e-5)
    g2 = g2_ref[...]
    be2 = be2_ref[...]
    w3t = w3_ref[...]
    b3 = b3_ref[...]
    gsel = (lax.broadcasted_iota(jnp.int32, (_RB, _RB // _K), 0)
            == _K * lax.broadcasted_iota(jnp.int32, (_RB, _RB // _K), 1)
            ).astype(jnp.float32)

    def l3_block(b, _):
        h2p = h2_ref[:, pl.ds(b * _RB, _RB)]
        h2 = jax.nn.relu((h2p - mu2) / den2 * g2 + be2)
        h3 = jnp.dot(w3t, h2, preferred_element_type=jnp.float32) + b3
        gmax = seg_max(h3)
        out_ref[b, :, :] = jnp.dot(
            gmax, gsel, preferred_element_type=jnp.float32)
        return 0

    lax.fori_loop(0, _NB, l3_block, 0)


def _mlp(rx, ry, W1, b1, g1, be1, W2, b2, g2, be2, W3, b3):
    vmem = pl.BlockSpec(memory_space=pltpu.VMEM)
    return pl.pallas_call(
        _mlp_body,
        out_shape=jax.ShapeDtypeStruct((_NB, 16, _RB // _K), jnp.float32),
        in_specs=[vmem] * 12,
        out_specs=vmem,
        scratch_shapes=[
            pltpu.VMEM((64, _N), jnp.float32),
            pltpu.VMEM((128, _N), jnp.float32),
        ],
    )(rx, ry, W1.T, b1.reshape(64, 1), g1.reshape(64, 1), be1.reshape(64, 1),
      W2.T, b2.reshape(128, 1), g2.reshape(128, 1), be2.reshape(128, 1),
      W3.T, b3.reshape(16, 1))


def kernel(points, W1, b1, g1, be1, W2, b2, g2, be2, W3, b3):
    pxf = points[:, 0]
    pyf = points[:, 1]
    px = pxf.reshape(_ROWS, _LANES)
    py = pyf.reshape(_ROWS, _LANES)
    cx, cy = _fps(px, py)
    cxs = cx.reshape(_CPW, _NWORK).T.reshape(_C)
    cys = cy.reshape(_CPW, _NWORK).T.reshape(_C)
    rxs, rys = _ballq_sc()(pxf, pyf, cxs, cys)
    relx = rxs.reshape(_NWORK, _CPW, _K).transpose(1, 0, 2).reshape(_N)
    rely = rys.reshape(_NWORK, _CPW, _K).transpose(1, 0, 2).reshape(_N)
    feats3 = _mlp(relx.reshape(1, _N), rely.reshape(1, _N),
                  W1, b1, g1, be1, W2, b2, g2, be2, W3, b3)
    feats = feats3.transpose(0, 2, 1).reshape(_C, 16)
    centroids = jnp.stack([cx, cy], axis=-1)
    return (feats, centroids)
